# baseline (speedup 1.0000x reference)
import functools

import jax
import jax.numpy as jnp
from jax import lax
from jax.experimental import pallas as pl
from jax.experimental.pallas import tpu as pltpu
from jax.experimental.pallas import tpu_sc as plsc

N = 10000
E = 320000
F = 128
H = 256
C = 10
G = 128

NC = 2
NS = 16
CH = 128
EPW = 20480
NSUB = EPW // CH
E_PAD = NS * EPW
N_ACC = 10240
RPT = N_ACC // NS
DW = 16
FH = F // NC

NBUF = 4
DEPTH = 2

R = 400
NB = N // R


def _sc_aggregate(xsp, es, ed):
    mesh = plsc.VectorSubcoreMesh(core_axis_name="c", subcore_axis_name="s")

    @functools.partial(
        pl.kernel,
        out_type=(
            jax.ShapeDtypeStruct((NC, N_ACC, FH), jnp.float32),
            jax.ShapeDtypeStruct((NC, N_ACC, DW), jnp.float32),
        ),
        mesh=mesh,
        scratch_types=[
            pltpu.VMEM((NSUB, CH), jnp.int32),
            pltpu.VMEM((NSUB, CH), jnp.int32),
            pltpu.VMEM((NBUF, CH, FH), jnp.float32),
            pltpu.VMEM((CH, DW), jnp.float32),
            pltpu.VMEM_SHARED((N_ACC, FH), jnp.float32),
            pltpu.VMEM_SHARED((N_ACC, DW), jnp.float32),
            [pltpu.SemaphoreType.DMA] * NBUF,
            [pltpu.SemaphoreType.DMA] * NBUF,
        ],
        compiler_params=pltpu.CompilerParams(use_tc_tiling_on_sc=False),
    )
    def k(x_hbm, es_hbm, ed_hbm, acc_out, deg_out, srcv, dstv, rows, ones,
          acc_sh, deg_sh, semg, sems):
        cid = lax.axis_index("c")
        sid = lax.axis_index("s")

        zero16 = jnp.zeros((16,), jnp.float32)
        one16 = jnp.ones((16,), jnp.float32)

        def zrow(i, _):
            for j in range(FH // 16):
                rows[0, i, pl.ds(j * 16, 16)] = zero16
            ones[i, pl.ds(0, DW)] = zero16
            return 0

        lax.fori_loop(0, CH, zrow, 0)

        for t in range(RPT // CH):
            r0 = sid * RPT + t * CH
            pltpu.sync_copy(rows.at[0], acc_sh.at[pl.ds(r0, CH)])
            pltpu.sync_copy(ones, deg_sh.at[pl.ds(r0, CH)])

        def orow(i, _):
            ones[i, pl.ds(0, DW)] = one16
            return 0

        lax.fori_loop(0, CH, orow, 0)

        pltpu.sync_copy(es_hbm.at[cid, sid], srcv)
        pltpu.sync_copy(ed_hbm.at[sid], dstv)

        plsc.subcore_barrier()

        def gather_start(j, b):
            pltpu.async_copy(x_hbm.at[srcv.at[j]], rows.at[b], semg[b])

        def gather_wait(j, b):
            pltpu.make_async_copy(x_hbm.at[srcv.at[j]], rows.at[b], semg[b]).wait()

        def scatter_start(j, b):
            pltpu.async_copy(rows.at[b], acc_sh.at[dstv.at[j]], sems[b], add=True)
            pltpu.async_copy(ones, deg_sh.at[dstv.at[j]], sems[b], add=True)

        def scatter_wait(j, b):
            pltpu.make_async_copy(rows.at[b], acc_sh.at[dstv.at[j]], sems[b]).wait()
            pltpu.make_async_copy(ones, deg_sh.at[dstv.at[j]], sems[b]).wait()

        for b0 in range(DEPTH):
            gather_start(b0, b0)

        def round_(jj, _):
            for b in range(NBUF):
                j = jj * NBUF + b
                gather_wait(j, b)
                scatter_start(j, b)

                @pl.when(j >= DEPTH)
                def _():
                    scatter_wait(j - DEPTH, (b - DEPTH) % NBUF)

                @pl.when(j + DEPTH < NSUB)
                def _():
                    gather_start(j + DEPTH, (b + DEPTH) % NBUF)
            return 0

        lax.fori_loop(0, NSUB // NBUF, round_, 0)
        for b0 in range(DEPTH):
            scatter_wait(NSUB - DEPTH + b0, (NSUB - DEPTH + b0) % NBUF)

        plsc.subcore_barrier()

        for t in range(RPT // CH):
            r0 = sid * RPT + t * CH
            pltpu.sync_copy(acc_sh.at[pl.ds(r0, CH)], acc_out.at[cid, pl.ds(r0, CH)])
            pltpu.sync_copy(deg_sh.at[pl.ds(r0, CH)], deg_out.at[cid, pl.ds(r0, CH)])

    return k(xsp, es, ed)


def _tc_body(acc_ref, deg_ref, x_ref, ids_ref, idsc_ref, wl_ref, bl_ref, wr_ref,
             w2_ref, b2_ref, out_ref, pooled):
    i = pl.program_id(0)
    a = jnp.concatenate([acc_ref[0], acc_ref[1]], axis=1)
    d = deg_ref[0, :, 0:1]
    agg = a / jnp.maximum(d, 1.0)
    h = jnp.dot(agg, wl_ref[...], preferred_element_type=jnp.float32)
    h += jnp.dot(x_ref[...], wr_ref[...], preferred_element_type=jnp.float32)
    h = jnp.maximum(h + bl_ref[...], 0.0)

    ids = ids_ref[0]
    idsc = idsc_ref[...]
    hh = h
    k = 1
    while k < R:
        same_col = (pltpu.roll(idsc, R - k, axis=0) == idsc)[:, 0:1]
        hh_s = pltpu.roll(hh, R - k, axis=0)
        hh = jnp.where(same_col, jnp.maximum(hh, hh_s), hh)
        k *= 2

    col = lax.broadcasted_iota(jnp.int32, (1, R), 1)
    prev = pltpu.roll(ids, 1, axis=1)
    is_start = (ids != prev) | (col == 0)
    gid = lax.broadcasted_iota(jnp.int32, (G, R), 0)
    eqg = ids == gid
    onehot = (eqg & is_start).astype(jnp.float32)
    contrib = jnp.dot(onehot, hh, preferred_element_type=jnp.float32)
    present = jnp.max(eqg.astype(jnp.float32), axis=1, keepdims=True)
    update = jnp.where(present > 0.0, contrib, -jnp.inf)

    @pl.when(i == 0)
    def _():
        pooled[...] = update

    @pl.when(i > 0)
    def _():
        pooled[...] = jnp.maximum(pooled[...], update)

    @pl.when(i == NB - 1)
    def _():
        out_ref[...] = (
            jnp.dot(pooled[...], w2_ref[...], preferred_element_type=jnp.float32)
            + b2_ref[...]
        )


def _tc_head(acc, deg, x_pad, ids3, idsc, wl, bl2, wr, w2p, b2p, interpret=False):
    return pl.pallas_call(
        _tc_body,
        grid=(NB,),
        in_specs=[
            pl.BlockSpec((NC, R, FH), lambda i: (0, i, 0)),
            pl.BlockSpec((NC, R, DW), lambda i: (0, i, 0)),
            pl.BlockSpec((R, F), lambda i: (i, 0)),
            pl.BlockSpec((1, 1, R), lambda i: (i, 0, 0)),
            pl.BlockSpec((R, 8), lambda i: (i, 0)),
            pl.BlockSpec((F, H), lambda i: (0, 0)),
            pl.BlockSpec((1, H), lambda i: (0, 0)),
            pl.BlockSpec((F, H), lambda i: (0, 0)),
            pl.BlockSpec((H, 128), lambda i: (0, 0)),
            pl.BlockSpec((1, 128), lambda i: (0, 0)),
        ],
        out_specs=pl.BlockSpec((G, 128), lambda i: (0, 0)),
        out_shape=jax.ShapeDtypeStruct((G, 128), jnp.float32),
        scratch_shapes=[pltpu.VMEM((G, H), jnp.float32)],
        interpret=interpret,
    )(acc, deg, x_pad, ids3, idsc, wl, bl2, wr, w2p, b2p)


@jax.jit
def kernel(x, edge_index, batch, Wl, bl, Wr, W2, b2):
    pad_e = E_PAD - E
    src_p = jnp.concatenate([edge_index[0], jnp.zeros((pad_e,), jnp.int32)])
    dst_p = jnp.concatenate([edge_index[1], jnp.full((pad_e,), N, jnp.int32)])
    es = jnp.stack([src_p, src_p + N]).reshape(NC, NS, NSUB, CH)
    ed = dst_p.reshape(NS, NSUB, CH)
    xsp = jnp.concatenate([x[:, :FH], x[:, FH:]], axis=0)

    acc, deg = _sc_aggregate(xsp, es, ed)

    ids3 = batch.reshape(NB, 1, R)
    idsc = jnp.broadcast_to(batch[:, None], (N, 8))
    bl2 = bl.reshape(1, H)
    w2p = jnp.zeros((H, 128), jnp.float32).at[:, :C].set(W2)
    b2p = jnp.zeros((1, 128), jnp.float32).at[0, :C].set(b2)

    out = _tc_head(acc, deg, x, ids3, idsc, Wl, bl2, Wr, w2p, b2p)
    return out[:, :C]


# speedup vs baseline: 1.0014x; 1.0014x over previous
import functools

import jax
import jax.numpy as jnp
from jax import lax
from jax.experimental import pallas as pl
from jax.experimental.pallas import tpu as pltpu
from jax.experimental.pallas import tpu_sc as plsc

N = 10000
E = 320000
F = 128
H = 256
C = 10
G = 128

NC = 2
NS = 16
CH = 128
EPW = 20480
NSUB = EPW // CH
E_PAD = NS * EPW
N_ACC = 10240
RPT = N_ACC // NS
DW = 16
FH = F // NC

NBUF = 4
DEPTH = 2

R = 400
NB = N // R


def _sc_aggregate(xsp, es, ed):
    mesh = plsc.VectorSubcoreMesh(core_axis_name="c", subcore_axis_name="s")

    @functools.partial(
        pl.kernel,
        out_type=(
            jax.ShapeDtypeStruct((NC, N_ACC, FH), jnp.float32),
            jax.ShapeDtypeStruct((NC, N_ACC, DW), jnp.float32),
        ),
        mesh=mesh,
        scratch_types=[
            pltpu.VMEM((NSUB, CH), jnp.int32),
            pltpu.VMEM((NSUB, CH), jnp.int32),
            pltpu.VMEM((NBUF, CH, FH), jnp.float32),
            pltpu.VMEM((CH, DW), jnp.float32),
            pltpu.VMEM_SHARED((N_ACC, FH), jnp.float32),
            pltpu.VMEM_SHARED((N_ACC, DW), jnp.float32),
            [pltpu.SemaphoreType.DMA] * NBUF,
            [pltpu.SemaphoreType.DMA] * NBUF,
        ],
        compiler_params=pltpu.CompilerParams(use_tc_tiling_on_sc=False),
    )
    def k(x_hbm, es_hbm, ed_hbm, acc_out, deg_out, srcv, dstv, rows, ones,
          acc_sh, deg_sh, semg, sems):
        cid = lax.axis_index("c")
        sid = lax.axis_index("s")

        zero16 = jnp.zeros((16,), jnp.float32)
        one16 = jnp.ones((16,), jnp.float32)

        def zrow(i, _):
            for j in range(FH // 16):
                rows[0, i, pl.ds(j * 16, 16)] = zero16
            ones[i, pl.ds(0, DW)] = zero16
            return 0

        lax.fori_loop(0, CH, zrow, 0)

        for t in range(RPT // CH):
            r0 = sid * RPT + t * CH
            pltpu.sync_copy(rows.at[0], acc_sh.at[pl.ds(r0, CH)])
            pltpu.sync_copy(ones, deg_sh.at[pl.ds(r0, CH)])

        def orow(i, _):
            ones[i, pl.ds(0, DW)] = one16
            return 0

        lax.fori_loop(0, CH, orow, 0)

        pltpu.sync_copy(es_hbm.at[cid, sid], srcv)
        pltpu.sync_copy(ed_hbm.at[sid], dstv)

        plsc.subcore_barrier()

        def gather_start(j, b):
            pltpu.async_copy(x_hbm.at[srcv.at[j]], rows.at[b], semg[b])

        def gather_wait(j, b):
            pltpu.make_async_copy(x_hbm.at[srcv.at[j]], rows.at[b], semg[b]).wait()

        def scatter_start(j, b):
            pltpu.async_copy(rows.at[b], acc_sh.at[dstv.at[j]], sems[b], add=True)
            pltpu.async_copy(ones, deg_sh.at[dstv.at[j]], sems[b], add=True)

        def scatter_wait(j, b):
            pltpu.make_async_copy(rows.at[b], acc_sh.at[dstv.at[j]], sems[b]).wait()
            pltpu.make_async_copy(ones, deg_sh.at[dstv.at[j]], sems[b]).wait()

        for b0 in range(DEPTH):
            gather_start(b0, b0)

        def round_(jj, _):
            for b in range(NBUF):
                j = jj * NBUF + b
                gather_wait(j, b)
                scatter_start(j, b)

                @pl.when(j >= DEPTH)
                def _():
                    scatter_wait(j - DEPTH, (b - DEPTH) % NBUF)

                @pl.when(j + DEPTH < NSUB)
                def _():
                    gather_start(j + DEPTH, (b + DEPTH) % NBUF)
            return 0

        lax.fori_loop(0, NSUB // NBUF, round_, 0)
        for b0 in range(DEPTH):
            scatter_wait(NSUB - DEPTH + b0, (NSUB - DEPTH + b0) % NBUF)

        plsc.subcore_barrier()

        for t in range(RPT // CH):
            r0 = sid * RPT + t * CH
            pltpu.sync_copy(acc_sh.at[pl.ds(r0, CH)], acc_out.at[cid, pl.ds(r0, CH)])
            pltpu.sync_copy(deg_sh.at[pl.ds(r0, CH)], deg_out.at[cid, pl.ds(r0, CH)])

    return k(xsp, es, ed)


def _tc_body(acc_ref, deg_ref, x_ref, ids_ref, idsc_ref, wl_ref, bl_ref, wr_ref,
             w2_ref, b2_ref, out_ref, pooled):
    i = pl.program_id(0)
    a = jnp.concatenate([acc_ref[0], acc_ref[1]], axis=1)
    d = deg_ref[0, :, 0:1]
    agg = a / jnp.maximum(d, 1.0)
    h = jnp.dot(agg, wl_ref[...], preferred_element_type=jnp.float32)
    h += jnp.dot(x_ref[...], wr_ref[...], preferred_element_type=jnp.float32)
    h = jnp.maximum(h + bl_ref[...], 0.0)

    ids = ids_ref[0]
    idsc = idsc_ref[...]
    hh = h
    k = 1
    while k < R:
        same_col = (pltpu.roll(idsc, R - k, axis=0) == idsc)[:, 0:1]
        hh_s = pltpu.roll(hh, R - k, axis=0)
        hh = jnp.where(same_col, jnp.maximum(hh, hh_s), hh)
        k *= 2

    col = lax.broadcasted_iota(jnp.int32, (1, R), 1)
    prev = pltpu.roll(ids, 1, axis=1)
    is_start = (ids != prev) | (col == 0)
    gid = lax.broadcasted_iota(jnp.int32, (G, R), 0)
    eqg = ids == gid
    onehot = (eqg & is_start).astype(jnp.float32)
    contrib = jnp.dot(onehot, hh, preferred_element_type=jnp.float32)
    present = jnp.max(eqg.astype(jnp.float32), axis=1, keepdims=True)
    update = jnp.where(present > 0.0, contrib, -jnp.inf)

    @pl.when(i == 0)
    def _():
        pooled[...] = update

    @pl.when(i > 0)
    def _():
        pooled[...] = jnp.maximum(pooled[...], update)

    @pl.when(i == NB - 1)
    def _():
        out_ref[...] = (
            jnp.dot(pooled[...], w2_ref[...], preferred_element_type=jnp.float32)
            + b2_ref[...]
        )


def _tc_head(acc, deg, x_pad, ids3, idsc, wl, bl2, wr, w2p, b2p):
    return pl.pallas_call(
        _tc_body,
        grid=(NB,),
        in_specs=[
            pl.BlockSpec((NC, R, FH), lambda i: (0, i, 0)),
            pl.BlockSpec((NC, R, DW), lambda i: (0, i, 0)),
            pl.BlockSpec((R, F), lambda i: (i, 0)),
            pl.BlockSpec((1, 1, R), lambda i: (i, 0, 0)),
            pl.BlockSpec((R, 8), lambda i: (i, 0)),
            pl.BlockSpec((F, H), lambda i: (0, 0)),
            pl.BlockSpec((1, H), lambda i: (0, 0)),
            pl.BlockSpec((F, H), lambda i: (0, 0)),
            pl.BlockSpec((H, 128), lambda i: (0, 0)),
            pl.BlockSpec((1, 128), lambda i: (0, 0)),
        ],
        out_specs=pl.BlockSpec((G, 128), lambda i: (0, 0)),
        out_shape=jax.ShapeDtypeStruct((G, 128), jnp.float32),
        scratch_shapes=[pltpu.VMEM((G, H), jnp.float32)],
    )(acc, deg, x_pad, ids3, idsc, wl, bl2, wr, w2p, b2p)


@jax.jit
def kernel(x, edge_index, batch, Wl, bl, Wr, W2, b2):
    pad_e = E_PAD - E
    src_p = jnp.concatenate([edge_index[0], jnp.zeros((pad_e,), jnp.int32)])
    dst_p = jnp.concatenate([edge_index[1], jnp.full((pad_e,), N, jnp.int32)])
    es = jnp.stack([src_p, src_p + N]).reshape(NC, NS, NSUB, CH)
    ed = dst_p.reshape(NS, NSUB, CH)
    xsp = jnp.concatenate([x[:, :FH], x[:, FH:]], axis=0)

    acc, deg = _sc_aggregate(xsp, es, ed)

    ids3 = batch.reshape(NB, 1, R)
    idsc = jnp.broadcast_to(batch[:, None], (N, 8))
    bl2 = bl.reshape(1, H)
    w2p = jnp.zeros((H, 128), jnp.float32).at[:, :C].set(W2)
    b2p = jnp.zeros((1, 128), jnp.float32).at[0, :C].set(b2)

    out = _tc_head(acc, deg, x, ids3, idsc, Wl, bl2, Wr, w2p, b2p)
    return out[:, :C]
